# baseline (speedup 1.0000x reference)
import dataclasses
import functools

import jax
import jax.numpy as jnp
import numpy as np
from jax import lax
from jax.experimental import pallas as pl
from jax.experimental.pallas import tpu as pltpu
from jax.experimental.pallas import tpu_sc as plsc

N = 10000
D = 128
E = 320000
NC = 2
NS = 16
NW = NC * NS
EPT = E // NW
NH = N // 2
NLOC = 5008
N_PAD = 10240
SCH = 64
NBK = 4
CAP = 10080
SUBROWS = 312


def _sc_compiler_params():
    cp = pltpu.CompilerParams()
    if "needs_layout_passes" in pltpu.CompilerParams.__dataclass_fields__:
        cp = dataclasses.replace(cp, needs_layout_passes=False)
    return cp


def _mesh():
    return plsc.VectorSubcoreMesh(
        core_axis_name="c", subcore_axis_name="s", num_cores=NC, num_subcores=NS
    )


def _deg_kernel_body(dst_hbm, out_hbm, dstv, counts):
    c = lax.axis_index("c")
    s = lax.axis_index("s")
    wid = s * NC + c
    pltpu.sync_copy(dst_hbm.at[pl.ds(wid * EPT, EPT)], dstv)

    @pl.loop(0, N, step=16)
    def _zero(i):
        counts[pl.ds(i, 16)] = jnp.zeros((16,), jnp.float32)

    ones = jnp.ones((16,), jnp.float32)

    @pl.loop(0, EPT, step=16)
    def _hist(i):
        idx = dstv[pl.ds(i, 16)]
        plsc.addupdate_scatter(counts, [idx], ones)

    pltpu.sync_copy(counts, out_hbm.at[wid])


@functools.lru_cache(maxsize=None)
def _deg_kernel_build():
    return pl.kernel(
        _deg_kernel_body,
        out_type=jax.ShapeDtypeStruct((NW, N), jnp.float32),
        mesh=_mesh(),
        scratch_types=[
            pltpu.VMEM((EPT,), jnp.int32),
            pltpu.VMEM((N,), jnp.float32),
        ],
        compiler_params=_sc_compiler_params(),
    )


def _deg_kernel(dst):
    return _deg_kernel_build()(dst)


def _part_kernel_body(src_hbm, dst_hbm, edges_hbm, counts_hbm,
                      srcv, dstv, bk0, bk1, bk2, bk3, cvv):
    buckets = (bk0, bk1, bk2, bk3)
    c = lax.axis_index("c")
    s = lax.axis_index("s")
    wid = s * NC + c
    pltpu.sync_copy(src_hbm.at[pl.ds(wid * EPT, EPT)], srcv)
    pltpu.sync_copy(dst_hbm.at[pl.ds(wid * EPT, EPT)], dstv)

    zero = jnp.int32(0)

    @pl.loop(0, EPT, step=16, init_carry=(zero, zero, zero, zero))
    def fills(i, carry):
        sv = srcv[pl.ds(i, 16)]
        dv = dstv[pl.ds(i, 16)]
        sh = jnp.where(sv >= NH, 1, 0).astype(jnp.int32)
        dh = jnp.where(dv >= NH, 1, 0).astype(jnp.int32)
        packed = ((sv - sh * NH) << 16) | (dv - dh * NH)
        bkt = dh * 2 + sh
        out = []
        for b in range(NBK):
            m = bkt == b
            plsc.store_compressed(buckets[b].at[pl.ds(carry[b], 16)],
                                  packed, mask=m)
            cnt = jnp.max(plsc.all_reduce_population_count(m))
            out.append(carry[b] + cnt)
        return tuple(out)

    pad = jnp.full((16,), (NH << 16) | NH, jnp.int32)
    lanes = lax.iota(jnp.int32, 16)
    cv = jnp.zeros((16,), jnp.int32)
    for b in range(NBK):
        f = fills[b]
        for q in range(SCH // 16):
            buckets[b][pl.ds(f + q * 16, 16)] = pad
        fpad = ((f + SCH - 1) // SCH) * SCH
        cv = jnp.where(lanes == b, fpad, cv)
    cvv[...] = cv
    pltpu.sync_copy(cvv, counts_hbm.at[pl.ds(wid * 16, 16)])
    for b in range(NBK):
        pltpu.sync_copy(buckets[b],
                        edges_hbm.at[pl.ds((wid * NBK + b) * CAP, CAP)])


@functools.lru_cache(maxsize=None)
def _part_kernel_build():
    return pl.kernel(
        _part_kernel_body,
        out_type=(jax.ShapeDtypeStruct((NW * NBK * CAP,), jnp.int32),
                  jax.ShapeDtypeStruct((NW * 16,), jnp.int32)),
        mesh=_mesh(),
        scratch_types=[
            pltpu.VMEM((EPT,), jnp.int32),
            pltpu.VMEM((EPT,), jnp.int32),
            pltpu.VMEM((CAP,), jnp.int32),
            pltpu.VMEM((CAP,), jnp.int32),
            pltpu.VMEM((CAP,), jnp.int32),
            pltpu.VMEM((CAP,), jnp.int32),
            pltpu.VMEM((16,), jnp.int32),
        ],
        compiler_params=_sc_compiler_params(),
    )


def _part_kernel(src, dst):
    return _part_kernel_build()(src, dst)


def _acc_kernel_body(hp_hbm, edges_hbm, counts_hbm, out_hbm,
                     pkv, sidx, didx, rows, cvm, hpl, acc, sem):
    c = lax.axis_index("c")
    s = lax.axis_index("s")
    lanes = lax.iota(jnp.int32, 16)
    ob = s * SUBROWS

    pltpu.sync_copy(counts_hbm, cvm)

    pltpu.sync_copy(hp_hbm.at[pl.ds(c * NH + ob, SUBROWS)],
                    acc.at[pl.ds(ob, SUBROWS)])

    @pl.when(s == 0)
    def _seed_tail():
        pltpu.sync_copy(hp_hbm.at[pl.ds(c * NH + NS * SUBROWS, NLOC - NS * SUBROWS)],
                        acc.at[pl.ds(NS * SUBROWS, NLOC - NS * SUBROWS)])

    for p in range(2):
        pltpu.sync_copy(hp_hbm.at[pl.ds(p * NH + ob, SUBROWS)],
                        hpl.at[pl.ds(ob, SUBROWS)])

        @pl.when(s == 0)
        def _stage_tail():
            pltpu.sync_copy(
                hp_hbm.at[pl.ds(p * NH + NS * SUBROWS, NLOC - NS * SUBROWS)],
                hpl.at[pl.ds(NS * SUBROWS, NLOC - NS * SUBROWS)])

        plsc.subcore_barrier()
        b = c * 2 + p
        for t_off in range(2):
            t = 2 * s + t_off
            crow = cvm[pl.ds(t * 16, 16)]
            cnt = jnp.max(jnp.where(lanes == b, crow, 0))
            regbase = (t * NBK + b) * CAP

            @pl.loop(0, cnt, step=SCH)
            def _chunk(k):
                pltpu.sync_copy(edges_hbm.at[pl.ds(regbase + k, SCH)], pkv)
                for q in range(SCH // 16):
                    pk = pkv[pl.ds(q * 16, 16)]
                    sidx[pl.ds(q * 16, 16)] = pk >> 16
                    didx[pl.ds(q * 16, 16)] = pk & 0xFFFF
                pltpu.async_copy(hpl.at[sidx], rows.at[0], sem).wait()
                pltpu.sync_copy(rows.at[0], acc.at[didx], add=True)

        plsc.subcore_barrier()

    pltpu.sync_copy(acc.at[pl.ds(ob, SUBROWS)],
                    out_hbm.at[c, pl.ds(ob, SUBROWS)])

    @pl.when(s == 0)
    def _out_tail():
        pltpu.sync_copy(acc.at[pl.ds(NS * SUBROWS, NLOC - NS * SUBROWS)],
                        out_hbm.at[c, pl.ds(NS * SUBROWS, NLOC - NS * SUBROWS)])


@functools.lru_cache(maxsize=None)
def _acc_kernel_build():
    return pl.kernel(
        _acc_kernel_body,
        out_type=jax.ShapeDtypeStruct((NC, NLOC, D), jnp.float32),
        mesh=_mesh(),
        scratch_types=[
            pltpu.VMEM((SCH,), jnp.int32),
            pltpu.VMEM((SCH,), jnp.int32),
            pltpu.VMEM((SCH,), jnp.int32),
            pltpu.VMEM((2, SCH, D), jnp.float32),
            pltpu.VMEM((NW * 16,), jnp.int32),
            pltpu.VMEM_SHARED((NLOC, D), jnp.float32),
            pltpu.VMEM_SHARED((NLOC, D), jnp.float32),
            pltpu.SemaphoreType.DMA,
        ],
        compiler_params=_sc_compiler_params(),
    )


def _acc_kernel(hp, edges, counts):
    return _acc_kernel_build()(hp, edges, counts)


_RB = 1000
_RBM = 2000


def _dinv_body(c_ref, o_ref):
    deg = jnp.sum(c_ref[...], axis=0, keepdims=True) + 1.0
    o_ref[...] = lax.rsqrt(deg)


def _mm_scale_body(x_ref, w_ref, dv_ref, o_ref):
    h = jnp.dot(x_ref[...], w_ref[...], preferred_element_type=jnp.float32)
    o_ref[...] = h * dv_ref[...]


def _ln_gelu(a, dv, b, g, be):
    y = a * dv + b
    mu = jnp.mean(y, axis=1, keepdims=True)
    d = y - mu
    var = jnp.mean(d * d, axis=1, keepdims=True)
    t = d * lax.rsqrt(var + 1e-5) * g + be
    return t * 0.5 * (1.0 + lax.erf(t * np.float32(1.0 / np.sqrt(2.0))))


def _comb_mm_body(acc_ref, dv_ref, b_ref, g_ref, be_ref, w_ref, o_ref):
    a = acc_ref[0]
    t = _ln_gelu(a, dv_ref[...], b_ref[...], g_ref[...], be_ref[...])
    h = jnp.dot(t, w_ref[...], preferred_element_type=jnp.float32)
    o_ref[...] = h * dv_ref[...]


def _comb_final_body(acc_ref, dv_ref, b_ref, g_ref, be_ref, o_ref):
    a = acc_ref[0]
    o_ref[...] = _ln_gelu(a, dv_ref[...], b_ref[...], g_ref[...], be_ref[...])


def _acc_spec():
    return pl.BlockSpec((1, _RB, D), lambda i: (i // 5, i % 5, 0))


def _dinv_call(counts):
    return pl.pallas_call(
        _dinv_body,
        out_shape=jax.ShapeDtypeStruct((1, N), jnp.float32),
    )(counts)


def _mm_scale_call(x, w, dv):
    return pl.pallas_call(
        _mm_scale_body,
        grid=(N // _RBM,),
        in_specs=[
            pl.BlockSpec((_RBM, D), lambda i: (i, 0)),
            pl.BlockSpec((D, D), lambda i: (0, 0)),
            pl.BlockSpec((_RBM, 1), lambda i: (i, 0)),
        ],
        out_specs=pl.BlockSpec((_RBM, D), lambda i: (i, 0)),
        out_shape=jax.ShapeDtypeStruct((N, D), jnp.float32),
    )(x, w, dv)


def _comb_mm_call(accs, dv, b, g, be, w):
    return pl.pallas_call(
        _comb_mm_body,
        grid=(N // _RB,),
        in_specs=[
            _acc_spec(),
            pl.BlockSpec((_RB, 1), lambda i: (i, 0)),
            pl.BlockSpec((1, D), lambda i: (0, 0)),
            pl.BlockSpec((1, D), lambda i: (0, 0)),
            pl.BlockSpec((1, D), lambda i: (0, 0)),
            pl.BlockSpec((D, D), lambda i: (0, 0)),
        ],
        out_specs=pl.BlockSpec((_RB, D), lambda i: (i, 0)),
        out_shape=jax.ShapeDtypeStruct((N, D), jnp.float32),
    )(accs, dv, b, g, be, w)


def _comb_final_call(accs, dv, b, g, be):
    return pl.pallas_call(
        _comb_final_body,
        grid=(N // _RB,),
        in_specs=[
            _acc_spec(),
            pl.BlockSpec((_RB, 1), lambda i: (i, 0)),
            pl.BlockSpec((1, D), lambda i: (0, 0)),
            pl.BlockSpec((1, D), lambda i: (0, 0)),
            pl.BlockSpec((1, D), lambda i: (0, 0)),
        ],
        out_specs=pl.BlockSpec((_RB, D), lambda i: (i, 0)),
        out_shape=jax.ShapeDtypeStruct((N, D), jnp.float32),
    )(accs, dv, b, g, be)


def _pad_rows(hp):
    return jnp.concatenate(
        [hp, jnp.zeros((N_PAD - N, D), jnp.float32)], axis=0)


def kernel(z, edge_index, W0, b0, W1, b1, W2, b2, g0, be0, g1, be1, g2, be2):
    src = edge_index[0]
    dst = edge_index[1]

    counts32 = _deg_kernel(dst)
    edges, counts = _part_kernel(src, dst)
    dv = _dinv_call(counts32).reshape(N, 1)

    b = [b0.reshape(1, D), b1.reshape(1, D), b2.reshape(1, D)]
    g = [g0.reshape(1, D), g1.reshape(1, D), g2.reshape(1, D)]
    be = [be0.reshape(1, D), be1.reshape(1, D), be2.reshape(1, D)]
    Wn = [W1, W2]

    hp = _pad_rows(_mm_scale_call(z, W0, dv))
    for i in range(3):
        accs = _acc_kernel(hp, edges, counts)
        if i < 2:
            hp = _pad_rows(_comb_mm_call(accs, dv, b[i], g[i], be[i], Wn[i]))
        else:
            out = _comb_final_call(accs, dv, b[i], g[i], be[i])
    return out


# speedup vs baseline: 1.2867x; 1.2867x over previous
import dataclasses
import functools

import jax
import jax.numpy as jnp
import numpy as np
from jax import lax
from jax.experimental import pallas as pl
from jax.experimental.pallas import tpu as pltpu
from jax.experimental.pallas import tpu_sc as plsc

N = 10000
D = 128
E = 320000
NC = 2
NS = 16
NW = NC * NS
EPT = E // NW
NH = N // 2
NLOC = 5008
N_PAD = 10240
SCH = 64
NBK = 4
CAP = 10240
SUBROWS = 312


def _sc_compiler_params():
    cp = pltpu.CompilerParams()
    if "needs_layout_passes" in pltpu.CompilerParams.__dataclass_fields__:
        cp = dataclasses.replace(cp, needs_layout_passes=False)
    return cp


def _mesh():
    return plsc.VectorSubcoreMesh(
        core_axis_name="c", subcore_axis_name="s", num_cores=NC, num_subcores=NS
    )


def _deg_kernel_body(dst_hbm, out_hbm, dstv, counts):
    c = lax.axis_index("c")
    s = lax.axis_index("s")
    wid = s * NC + c
    pltpu.sync_copy(dst_hbm.at[pl.ds(wid * EPT, EPT)], dstv)

    @pl.loop(0, N, step=16)
    def _zero(i):
        counts[pl.ds(i, 16)] = jnp.zeros((16,), jnp.float32)

    ones = jnp.ones((16,), jnp.float32)

    @pl.loop(0, EPT, step=16)
    def _hist(i):
        idx = dstv[pl.ds(i, 16)]
        plsc.addupdate_scatter(counts, [idx], ones)

    pltpu.sync_copy(counts, out_hbm.at[wid])


@functools.lru_cache(maxsize=None)
def _deg_kernel_build():
    return pl.kernel(
        _deg_kernel_body,
        out_type=jax.ShapeDtypeStruct((NW, N), jnp.float32),
        mesh=_mesh(),
        scratch_types=[
            pltpu.VMEM((EPT,), jnp.int32),
            pltpu.VMEM((N,), jnp.float32),
        ],
        compiler_params=_sc_compiler_params(),
    )


def _deg_kernel(dst):
    return _deg_kernel_build()(dst)


def _part_kernel_body(src_hbm, dst_hbm, edges_hbm, counts_hbm,
                      srcv, dstv, bk0, bk1, bk2, bk3, cvv):
    buckets = (bk0, bk1, bk2, bk3)
    c = lax.axis_index("c")
    s = lax.axis_index("s")
    wid = s * NC + c
    pltpu.sync_copy(src_hbm.at[pl.ds(wid * EPT, EPT)], srcv)
    pltpu.sync_copy(dst_hbm.at[pl.ds(wid * EPT, EPT)], dstv)

    zero = jnp.int32(0)

    @pl.loop(0, EPT, step=16, init_carry=(zero, zero, zero, zero))
    def fills(i, carry):
        sv = srcv[pl.ds(i, 16)]
        dv = dstv[pl.ds(i, 16)]
        sh = jnp.where(sv >= NH, 1, 0).astype(jnp.int32)
        dh = jnp.where(dv >= NH, 1, 0).astype(jnp.int32)
        packed = ((sv - sh * NH) << 16) | (dv - dh * NH)
        bkt = dh * 2 + sh
        out = []
        for b in range(NBK):
            m = bkt == b
            plsc.store_compressed(buckets[b].at[pl.ds(carry[b], 16)],
                                  packed, mask=m)
            cnt = jnp.max(plsc.all_reduce_population_count(m))
            out.append(carry[b] + cnt)
        return tuple(out)

    pad = jnp.full((16,), (NH << 16) | NH, jnp.int32)
    lanes = lax.iota(jnp.int32, 16)
    cv = jnp.zeros((16,), jnp.int32)
    for b in range(NBK):
        f = fills[b]
        for q in range(2 * SCH // 16):
            buckets[b][pl.ds(f + q * 16, 16)] = pad
        fpad = ((f + 2 * SCH - 1) // (2 * SCH)) * (2 * SCH)
        cv = jnp.where(lanes == b, fpad, cv)
    cvv[...] = cv
    pltpu.sync_copy(cvv, counts_hbm.at[pl.ds(wid * 16, 16)])
    for b in range(NBK):
        pltpu.sync_copy(buckets[b],
                        edges_hbm.at[pl.ds((wid * NBK + b) * CAP, CAP)])


@functools.lru_cache(maxsize=None)
def _part_kernel_build():
    return pl.kernel(
        _part_kernel_body,
        out_type=(jax.ShapeDtypeStruct((NW * NBK * CAP,), jnp.int32),
                  jax.ShapeDtypeStruct((NW * 16,), jnp.int32)),
        mesh=_mesh(),
        scratch_types=[
            pltpu.VMEM((EPT,), jnp.int32),
            pltpu.VMEM((EPT,), jnp.int32),
            pltpu.VMEM((CAP,), jnp.int32),
            pltpu.VMEM((CAP,), jnp.int32),
            pltpu.VMEM((CAP,), jnp.int32),
            pltpu.VMEM((CAP,), jnp.int32),
            pltpu.VMEM((16,), jnp.int32),
        ],
        compiler_params=_sc_compiler_params(),
    )


def _part_kernel(src, dst):
    return _part_kernel_build()(src, dst)


def _acc_kernel_body(hp_hbm, edges_hbm, counts_hbm, out_hbm,
                     pkvA, pkvB, sidxA, sidxB, didxA, didxB, rows, cvm,
                     hpl, acc, sem, sem_i):
    c = lax.axis_index("c")
    s = lax.axis_index("s")
    lanes = lax.iota(jnp.int32, 16)
    ob = s * SUBROWS

    pltpu.sync_copy(counts_hbm, cvm)

    pltpu.sync_copy(hp_hbm.at[pl.ds(c * NH + ob, SUBROWS)],
                    acc.at[pl.ds(ob, SUBROWS)])

    @pl.when(s == 0)
    def _seed_tail():
        pltpu.sync_copy(hp_hbm.at[pl.ds(c * NH + NS * SUBROWS, NLOC - NS * SUBROWS)],
                        acc.at[pl.ds(NS * SUBROWS, NLOC - NS * SUBROWS)])

    for p in range(2):
        pltpu.sync_copy(hp_hbm.at[pl.ds(p * NH + ob, SUBROWS)],
                        hpl.at[pl.ds(ob, SUBROWS)])

        @pl.when(s == 0)
        def _stage_tail():
            pltpu.sync_copy(
                hp_hbm.at[pl.ds(p * NH + NS * SUBROWS, NLOC - NS * SUBROWS)],
                hpl.at[pl.ds(NS * SUBROWS, NLOC - NS * SUBROWS)])

        plsc.subcore_barrier()
        b = c * 2 + p
        for t_off in range(2):
            t = 2 * s + t_off
            crow = cvm[pl.ds(t * 16, 16)]
            cnt = jnp.max(jnp.where(lanes == b, crow, 0))
            regbase = (t * NBK + b) * CAP

            def idx_load(off, buf):
                return pltpu.async_copy(
                    edges_hbm.at[pl.ds(regbase + off, SCH)], buf, sem_i)

            def unpack(pkv, sidx, didx):
                for q in range(SCH // 16):
                    pk = pkv[pl.ds(q * 16, 16)]
                    sidx[pl.ds(q * 16, 16)] = pk >> 16
                    didx[pl.ds(q * 16, 16)] = pk & 0xFFFF

            idx_load(0, pkvA)
            idx_load(SCH, pkvB)

            @pl.loop(0, cnt, step=2 * SCH)
            def _pair(k):
                pltpu.make_async_copy(edges_hbm.at[pl.ds(regbase + k, SCH)],
                                      pkvA, sem_i).wait()
                unpack(pkvA, sidxA, didxA)
                pltpu.async_copy(hpl.at[sidxA], rows.at[0], sem)
                idx_load(k + 2 * SCH, pkvA)
                pltpu.make_async_copy(
                    edges_hbm.at[pl.ds(regbase + k + SCH, SCH)],
                    pkvB, sem_i).wait()
                unpack(pkvB, sidxB, didxB)
                pltpu.make_async_copy(hpl.at[sidxA], rows.at[0], sem).wait()
                pltpu.async_copy(hpl.at[sidxB], rows.at[1], sem)
                pltpu.sync_copy(rows.at[0], acc.at[didxA], add=True)
                idx_load(k + 3 * SCH, pkvB)
                pltpu.make_async_copy(hpl.at[sidxB], rows.at[1], sem).wait()
                pltpu.sync_copy(rows.at[1], acc.at[didxB], add=True)

            pltpu.make_async_copy(edges_hbm.at[pl.ds(regbase, SCH)],
                                  pkvA, sem_i).wait()
            pltpu.make_async_copy(edges_hbm.at[pl.ds(regbase, SCH)],
                                  pkvB, sem_i).wait()

        plsc.subcore_barrier()

    pltpu.sync_copy(acc.at[pl.ds(ob, SUBROWS)],
                    out_hbm.at[c, pl.ds(ob, SUBROWS)])

    @pl.when(s == 0)
    def _out_tail():
        pltpu.sync_copy(acc.at[pl.ds(NS * SUBROWS, NLOC - NS * SUBROWS)],
                        out_hbm.at[c, pl.ds(NS * SUBROWS, NLOC - NS * SUBROWS)])


@functools.lru_cache(maxsize=None)
def _acc_kernel_build():
    return pl.kernel(
        _acc_kernel_body,
        out_type=jax.ShapeDtypeStruct((NC, NLOC, D), jnp.float32),
        mesh=_mesh(),
        scratch_types=[
            pltpu.VMEM((SCH,), jnp.int32),
            pltpu.VMEM((SCH,), jnp.int32),
            pltpu.VMEM((SCH,), jnp.int32),
            pltpu.VMEM((SCH,), jnp.int32),
            pltpu.VMEM((SCH,), jnp.int32),
            pltpu.VMEM((SCH,), jnp.int32),
            pltpu.VMEM((2, SCH, D), jnp.float32),
            pltpu.VMEM((NW * 16,), jnp.int32),
            pltpu.VMEM_SHARED((NLOC, D), jnp.float32),
            pltpu.VMEM_SHARED((NLOC, D), jnp.float32),
            pltpu.SemaphoreType.DMA,
            pltpu.SemaphoreType.DMA,
        ],
        compiler_params=_sc_compiler_params(),
    )


def _acc_kernel(hp, edges, counts):
    return _acc_kernel_build()(hp, edges, counts)


_RB = 1000
_RBM = 2000


def _dinv_body(c_ref, o_ref):
    deg = jnp.sum(c_ref[...], axis=0, keepdims=True) + 1.0
    o_ref[...] = lax.rsqrt(deg)


def _mm_scale_body(x_ref, w_ref, dv_ref, o_ref):
    h = jnp.dot(x_ref[...], w_ref[...], preferred_element_type=jnp.float32)
    o_ref[...] = h * dv_ref[...]


def _ln_gelu(a, dv, b, g, be):
    y = a * dv + b
    mu = jnp.mean(y, axis=1, keepdims=True)
    d = y - mu
    var = jnp.mean(d * d, axis=1, keepdims=True)
    t = d * lax.rsqrt(var + 1e-5) * g + be
    return t * 0.5 * (1.0 + lax.erf(t * np.float32(1.0 / np.sqrt(2.0))))


def _comb_mm_body(acc_ref, dv_ref, b_ref, g_ref, be_ref, w_ref, o_ref):
    a = acc_ref[0]
    t = _ln_gelu(a, dv_ref[...], b_ref[...], g_ref[...], be_ref[...])
    h = jnp.dot(t, w_ref[...], preferred_element_type=jnp.float32)
    o_ref[...] = h * dv_ref[...]


def _comb_final_body(acc_ref, dv_ref, b_ref, g_ref, be_ref, o_ref):
    a = acc_ref[0]
    o_ref[...] = _ln_gelu(a, dv_ref[...], b_ref[...], g_ref[...], be_ref[...])


def _acc_spec():
    return pl.BlockSpec((1, _RB, D), lambda i: (i // 5, i % 5, 0))


def _dinv_call(counts):
    return pl.pallas_call(
        _dinv_body,
        out_shape=jax.ShapeDtypeStruct((1, N), jnp.float32),
    )(counts)


def _mm_scale_call(x, w, dv):
    return pl.pallas_call(
        _mm_scale_body,
        grid=(N // _RBM,),
        in_specs=[
            pl.BlockSpec((_RBM, D), lambda i: (i, 0)),
            pl.BlockSpec((D, D), lambda i: (0, 0)),
            pl.BlockSpec((_RBM, 1), lambda i: (i, 0)),
        ],
        out_specs=pl.BlockSpec((_RBM, D), lambda i: (i, 0)),
        out_shape=jax.ShapeDtypeStruct((N, D), jnp.float32),
    )(x, w, dv)


def _comb_mm_call(accs, dv, b, g, be, w):
    return pl.pallas_call(
        _comb_mm_body,
        grid=(N // _RB,),
        in_specs=[
            _acc_spec(),
            pl.BlockSpec((_RB, 1), lambda i: (i, 0)),
            pl.BlockSpec((1, D), lambda i: (0, 0)),
            pl.BlockSpec((1, D), lambda i: (0, 0)),
            pl.BlockSpec((1, D), lambda i: (0, 0)),
            pl.BlockSpec((D, D), lambda i: (0, 0)),
        ],
        out_specs=pl.BlockSpec((_RB, D), lambda i: (i, 0)),
        out_shape=jax.ShapeDtypeStruct((N, D), jnp.float32),
    )(accs, dv, b, g, be, w)


def _comb_final_call(accs, dv, b, g, be):
    return pl.pallas_call(
        _comb_final_body,
        grid=(N // _RB,),
        in_specs=[
            _acc_spec(),
            pl.BlockSpec((_RB, 1), lambda i: (i, 0)),
            pl.BlockSpec((1, D), lambda i: (0, 0)),
            pl.BlockSpec((1, D), lambda i: (0, 0)),
            pl.BlockSpec((1, D), lambda i: (0, 0)),
        ],
        out_specs=pl.BlockSpec((_RB, D), lambda i: (i, 0)),
        out_shape=jax.ShapeDtypeStruct((N, D), jnp.float32),
    )(accs, dv, b, g, be)


def _pad_rows(hp):
    return jnp.concatenate(
        [hp, jnp.zeros((N_PAD - N, D), jnp.float32)], axis=0)


def kernel(z, edge_index, W0, b0, W1, b1, W2, b2, g0, be0, g1, be1, g2, be2):
    src = edge_index[0]
    dst = edge_index[1]

    counts32 = _deg_kernel(dst)
    edges, counts = _part_kernel(src, dst)
    dv = _dinv_call(counts32).reshape(N, 1)

    b = [b0.reshape(1, D), b1.reshape(1, D), b2.reshape(1, D)]
    g = [g0.reshape(1, D), g1.reshape(1, D), g2.reshape(1, D)]
    be = [be0.reshape(1, D), be1.reshape(1, D), be2.reshape(1, D)]
    Wn = [W1, W2]

    hp = _pad_rows(_mm_scale_call(z, W0, dv))
    for i in range(3):
        accs = _acc_kernel(hp, edges, counts)
        if i < 2:
            hp = _pad_rows(_comb_mm_call(accs, dv, b[i], g[i], be[i], Wn[i]))
        else:
            out = _comb_final_call(accs, dv, b[i], g[i], be[i])
    return out


# speedup vs baseline: 1.4045x; 1.0916x over previous
import dataclasses
import functools

import jax
import jax.numpy as jnp
import numpy as np
from jax import lax
from jax.experimental import pallas as pl
from jax.experimental.pallas import tpu as pltpu
from jax.experimental.pallas import tpu_sc as plsc

N = 10000
D = 128
E = 320000
NC = 2
NS = 16
NW = NC * NS
EPT = E // NW
NH = N // 2
NLOC = 5008
N_PAD = 10240
SCH = 128
NBK = 4
CAP = 10496
SUBROWS = 312


def _sc_compiler_params():
    cp = pltpu.CompilerParams()
    if "needs_layout_passes" in pltpu.CompilerParams.__dataclass_fields__:
        cp = dataclasses.replace(cp, needs_layout_passes=False)
    return cp


def _mesh():
    return plsc.VectorSubcoreMesh(
        core_axis_name="c", subcore_axis_name="s", num_cores=NC, num_subcores=NS
    )


def _deg_kernel_body(dst_hbm, out_hbm, dstv, counts):
    c = lax.axis_index("c")
    s = lax.axis_index("s")
    wid = s * NC + c
    pltpu.sync_copy(dst_hbm.at[pl.ds(wid * EPT, EPT)], dstv)

    @pl.loop(0, N, step=16)
    def _zero(i):
        counts[pl.ds(i, 16)] = jnp.zeros((16,), jnp.float32)

    ones = jnp.ones((16,), jnp.float32)

    @pl.loop(0, EPT, step=16)
    def _hist(i):
        idx = dstv[pl.ds(i, 16)]
        plsc.addupdate_scatter(counts, [idx], ones)

    pltpu.sync_copy(counts, out_hbm.at[wid])


@functools.lru_cache(maxsize=None)
def _deg_kernel_build():
    return pl.kernel(
        _deg_kernel_body,
        out_type=jax.ShapeDtypeStruct((NW, N), jnp.float32),
        mesh=_mesh(),
        scratch_types=[
            pltpu.VMEM((EPT,), jnp.int32),
            pltpu.VMEM((N,), jnp.float32),
        ],
        compiler_params=_sc_compiler_params(),
    )


def _deg_kernel(dst):
    return _deg_kernel_build()(dst)


def _part_kernel_body(src_hbm, dst_hbm, edges_hbm, counts_hbm,
                      srcv, dstv, bk0, bk1, bk2, bk3, cvv):
    buckets = (bk0, bk1, bk2, bk3)
    c = lax.axis_index("c")
    s = lax.axis_index("s")
    wid = s * NC + c
    pltpu.sync_copy(src_hbm.at[pl.ds(wid * EPT, EPT)], srcv)
    pltpu.sync_copy(dst_hbm.at[pl.ds(wid * EPT, EPT)], dstv)

    zero = jnp.int32(0)

    @pl.loop(0, EPT, step=16, init_carry=(zero, zero, zero, zero))
    def fills(i, carry):
        sv = srcv[pl.ds(i, 16)]
        dv = dstv[pl.ds(i, 16)]
        sh = jnp.where(sv >= NH, 1, 0).astype(jnp.int32)
        dh = jnp.where(dv >= NH, 1, 0).astype(jnp.int32)
        packed = ((sv - sh * NH) << 16) | (dv - dh * NH)
        bkt = dh * 2 + sh
        out = []
        for b in range(NBK):
            m = bkt == b
            plsc.store_compressed(buckets[b].at[pl.ds(carry[b], 16)],
                                  packed, mask=m)
            cnt = jnp.max(plsc.all_reduce_population_count(m))
            out.append(carry[b] + cnt)
        return tuple(out)

    pad = jnp.full((16,), (NH << 16) | NH, jnp.int32)
    lanes = lax.iota(jnp.int32, 16)
    cv = jnp.zeros((16,), jnp.int32)
    for b in range(NBK):
        f = fills[b]
        for q in range(2 * SCH // 16):
            buckets[b][pl.ds(f + q * 16, 16)] = pad
        fpad = ((f + 2 * SCH - 1) // (2 * SCH)) * (2 * SCH)
        cv = jnp.where(lanes == b, fpad, cv)
    cvv[...] = cv
    pltpu.sync_copy(cvv, counts_hbm.at[pl.ds(wid * 16, 16)])
    for b in range(NBK):
        pltpu.sync_copy(buckets[b],
                        edges_hbm.at[pl.ds((wid * NBK + b) * CAP, CAP)])


@functools.lru_cache(maxsize=None)
def _part_kernel_build():
    return pl.kernel(
        _part_kernel_body,
        out_type=(jax.ShapeDtypeStruct((NW * NBK * CAP,), jnp.int32),
                  jax.ShapeDtypeStruct((NW * 16,), jnp.int32)),
        mesh=_mesh(),
        scratch_types=[
            pltpu.VMEM((EPT,), jnp.int32),
            pltpu.VMEM((EPT,), jnp.int32),
            pltpu.VMEM((CAP,), jnp.int32),
            pltpu.VMEM((CAP,), jnp.int32),
            pltpu.VMEM((CAP,), jnp.int32),
            pltpu.VMEM((CAP,), jnp.int32),
            pltpu.VMEM((16,), jnp.int32),
        ],
        compiler_params=_sc_compiler_params(),
    )


def _part_kernel(src, dst):
    return _part_kernel_build()(src, dst)


def _acc_kernel_body(hp_hbm, edges_hbm, counts_hbm, out_hbm,
                     pkvA, pkvB, sidxA, sidxB, didxA, didxB, rows, cvm,
                     hpl, acc, sem, sem_i):
    c = lax.axis_index("c")
    s = lax.axis_index("s")
    lanes = lax.iota(jnp.int32, 16)
    ob = s * SUBROWS

    pltpu.sync_copy(counts_hbm, cvm)

    pltpu.sync_copy(hp_hbm.at[pl.ds(c * NH + ob, SUBROWS)],
                    acc.at[pl.ds(ob, SUBROWS)])

    @pl.when(s == 0)
    def _seed_tail():
        pltpu.sync_copy(hp_hbm.at[pl.ds(c * NH + NS * SUBROWS, NLOC - NS * SUBROWS)],
                        acc.at[pl.ds(NS * SUBROWS, NLOC - NS * SUBROWS)])

    for p in range(2):
        pltpu.sync_copy(hp_hbm.at[pl.ds(p * NH + ob, SUBROWS)],
                        hpl.at[pl.ds(ob, SUBROWS)])

        @pl.when(s == 0)
        def _stage_tail():
            pltpu.sync_copy(
                hp_hbm.at[pl.ds(p * NH + NS * SUBROWS, NLOC - NS * SUBROWS)],
                hpl.at[pl.ds(NS * SUBROWS, NLOC - NS * SUBROWS)])

        plsc.subcore_barrier()
        b = c * 2 + p
        for t_off in range(2):
            t = 2 * s + t_off
            crow = cvm[pl.ds(t * 16, 16)]
            cnt = jnp.max(jnp.where(lanes == b, crow, 0))
            regbase = (t * NBK + b) * CAP

            def idx_load(off, buf):
                return pltpu.async_copy(
                    edges_hbm.at[pl.ds(regbase + off, SCH)], buf, sem_i)

            def unpack(pkv, sidx, didx):
                for q in range(SCH // 16):
                    pk = pkv[pl.ds(q * 16, 16)]
                    sidx[pl.ds(q * 16, 16)] = pk >> 16
                    didx[pl.ds(q * 16, 16)] = pk & 0xFFFF

            idx_load(0, pkvA)
            idx_load(SCH, pkvB)

            @pl.loop(0, cnt, step=2 * SCH)
            def _pair(k):
                pltpu.make_async_copy(edges_hbm.at[pl.ds(regbase + k, SCH)],
                                      pkvA, sem_i).wait()
                unpack(pkvA, sidxA, didxA)
                pltpu.async_copy(hpl.at[sidxA], rows.at[0], sem)
                idx_load(k + 2 * SCH, pkvA)
                pltpu.make_async_copy(
                    edges_hbm.at[pl.ds(regbase + k + SCH, SCH)],
                    pkvB, sem_i).wait()
                unpack(pkvB, sidxB, didxB)
                pltpu.make_async_copy(hpl.at[sidxA], rows.at[0], sem).wait()
                pltpu.async_copy(hpl.at[sidxB], rows.at[1], sem)
                pltpu.sync_copy(rows.at[0], acc.at[didxA], add=True)
                idx_load(k + 3 * SCH, pkvB)
                pltpu.make_async_copy(hpl.at[sidxB], rows.at[1], sem).wait()
                pltpu.sync_copy(rows.at[1], acc.at[didxB], add=True)

            pltpu.make_async_copy(edges_hbm.at[pl.ds(regbase, SCH)],
                                  pkvA, sem_i).wait()
            pltpu.make_async_copy(edges_hbm.at[pl.ds(regbase, SCH)],
                                  pkvB, sem_i).wait()

        plsc.subcore_barrier()

    pltpu.sync_copy(acc.at[pl.ds(ob, SUBROWS)],
                    out_hbm.at[c, pl.ds(ob, SUBROWS)])

    @pl.when(s == 0)
    def _out_tail():
        pltpu.sync_copy(acc.at[pl.ds(NS * SUBROWS, NLOC - NS * SUBROWS)],
                        out_hbm.at[c, pl.ds(NS * SUBROWS, NLOC - NS * SUBROWS)])


@functools.lru_cache(maxsize=None)
def _acc_kernel_build():
    return pl.kernel(
        _acc_kernel_body,
        out_type=jax.ShapeDtypeStruct((NC, NLOC, D), jnp.float32),
        mesh=_mesh(),
        scratch_types=[
            pltpu.VMEM((SCH,), jnp.int32),
            pltpu.VMEM((SCH,), jnp.int32),
            pltpu.VMEM((SCH,), jnp.int32),
            pltpu.VMEM((SCH,), jnp.int32),
            pltpu.VMEM((SCH,), jnp.int32),
            pltpu.VMEM((SCH,), jnp.int32),
            pltpu.VMEM((2, SCH, D), jnp.float32),
            pltpu.VMEM((NW * 16,), jnp.int32),
            pltpu.VMEM_SHARED((NLOC, D), jnp.float32),
            pltpu.VMEM_SHARED((NLOC, D), jnp.float32),
            pltpu.SemaphoreType.DMA,
            pltpu.SemaphoreType.DMA,
        ],
        compiler_params=_sc_compiler_params(),
    )


def _acc_kernel(hp, edges, counts):
    return _acc_kernel_build()(hp, edges, counts)


_RB = 1000
_RBM = 2000


def _dinv_body(c_ref, o_ref):
    deg = jnp.sum(c_ref[...], axis=0, keepdims=True) + 1.0
    o_ref[...] = lax.rsqrt(deg)


def _mm_scale_body(x_ref, w_ref, dv_ref, o_ref):
    h = jnp.dot(x_ref[...], w_ref[...], preferred_element_type=jnp.float32)
    o_ref[...] = h * dv_ref[...]


def _ln_gelu(a, dv, b, g, be):
    y = a * dv + b
    mu = jnp.mean(y, axis=1, keepdims=True)
    d = y - mu
    var = jnp.mean(d * d, axis=1, keepdims=True)
    t = d * lax.rsqrt(var + 1e-5) * g + be
    return t * 0.5 * (1.0 + lax.erf(t * np.float32(1.0 / np.sqrt(2.0))))


def _comb_mm_body(acc_ref, dv_ref, b_ref, g_ref, be_ref, w_ref, o_ref):
    a = acc_ref[0]
    t = _ln_gelu(a, dv_ref[...], b_ref[...], g_ref[...], be_ref[...])
    h = jnp.dot(t, w_ref[...], preferred_element_type=jnp.float32)
    o_ref[...] = h * dv_ref[...]


def _comb_final_body(acc_ref, dv_ref, b_ref, g_ref, be_ref, o_ref):
    a = acc_ref[0]
    o_ref[...] = _ln_gelu(a, dv_ref[...], b_ref[...], g_ref[...], be_ref[...])


def _acc_spec():
    return pl.BlockSpec((1, _RB, D), lambda i: (i // 5, i % 5, 0))


def _dinv_call(counts):
    return pl.pallas_call(
        _dinv_body,
        out_shape=jax.ShapeDtypeStruct((1, N), jnp.float32),
    )(counts)


def _mm_scale_call(x, w, dv):
    return pl.pallas_call(
        _mm_scale_body,
        grid=(N // _RBM,),
        in_specs=[
            pl.BlockSpec((_RBM, D), lambda i: (i, 0)),
            pl.BlockSpec((D, D), lambda i: (0, 0)),
            pl.BlockSpec((_RBM, 1), lambda i: (i, 0)),
        ],
        out_specs=pl.BlockSpec((_RBM, D), lambda i: (i, 0)),
        out_shape=jax.ShapeDtypeStruct((N_PAD, D), jnp.float32),
    )(x, w, dv)


def _comb_mm_call(accs, dv, b, g, be, w):
    return pl.pallas_call(
        _comb_mm_body,
        grid=(N // _RB,),
        in_specs=[
            _acc_spec(),
            pl.BlockSpec((_RB, 1), lambda i: (i, 0)),
            pl.BlockSpec((1, D), lambda i: (0, 0)),
            pl.BlockSpec((1, D), lambda i: (0, 0)),
            pl.BlockSpec((1, D), lambda i: (0, 0)),
            pl.BlockSpec((D, D), lambda i: (0, 0)),
        ],
        out_specs=pl.BlockSpec((_RB, D), lambda i: (i, 0)),
        out_shape=jax.ShapeDtypeStruct((N_PAD, D), jnp.float32),
    )(accs, dv, b, g, be, w)


def _comb_final_call(accs, dv, b, g, be):
    return pl.pallas_call(
        _comb_final_body,
        grid=(N // _RB,),
        in_specs=[
            _acc_spec(),
            pl.BlockSpec((_RB, 1), lambda i: (i, 0)),
            pl.BlockSpec((1, D), lambda i: (0, 0)),
            pl.BlockSpec((1, D), lambda i: (0, 0)),
            pl.BlockSpec((1, D), lambda i: (0, 0)),
        ],
        out_specs=pl.BlockSpec((_RB, D), lambda i: (i, 0)),
        out_shape=jax.ShapeDtypeStruct((N, D), jnp.float32),
    )(accs, dv, b, g, be)


def kernel(z, edge_index, W0, b0, W1, b1, W2, b2, g0, be0, g1, be1, g2, be2):
    src = edge_index[0]
    dst = edge_index[1]

    counts32 = _deg_kernel(dst)
    edges, counts = _part_kernel(src, dst)
    dv = _dinv_call(counts32).reshape(N, 1)

    b = [b0.reshape(1, D), b1.reshape(1, D), b2.reshape(1, D)]
    g = [g0.reshape(1, D), g1.reshape(1, D), g2.reshape(1, D)]
    be = [be0.reshape(1, D), be1.reshape(1, D), be2.reshape(1, D)]
    Wn = [W1, W2]

    hp = _mm_scale_call(z, W0, dv)
    for i in range(3):
        accs = _acc_kernel(hp, edges, counts)
        if i < 2:
            hp = _comb_mm_call(accs, dv, b[i], g[i], be[i], Wn[i])
        else:
            out = _comb_final_call(accs, dv, b[i], g[i], be[i])
    return out


# speedup vs baseline: 1.4316x; 1.0192x over previous
import dataclasses
import functools

import jax
import jax.numpy as jnp
import numpy as np
from jax import lax
from jax.experimental import pallas as pl
from jax.experimental.pallas import tpu as pltpu
from jax.experimental.pallas import tpu_sc as plsc

N = 10000
D = 128
E = 320000
NC = 2
NS = 16
NW = NC * NS
EPT = E // NW
NH = N // 2
NLOC = 5008
N_PAD = 10240
SCH = 128
NBK = 4
CAP = 10496
SUBROWS = 312


def _sc_compiler_params():
    cp = pltpu.CompilerParams()
    if "needs_layout_passes" in pltpu.CompilerParams.__dataclass_fields__:
        cp = dataclasses.replace(cp, needs_layout_passes=False)
    return cp


def _mesh():
    return plsc.VectorSubcoreMesh(
        core_axis_name="c", subcore_axis_name="s", num_cores=NC, num_subcores=NS
    )


def _deg_kernel_body(dst_hbm, out_hbm, dstv, counts):
    c = lax.axis_index("c")
    s = lax.axis_index("s")
    wid = s * NC + c
    pltpu.sync_copy(dst_hbm.at[pl.ds(wid * EPT, EPT)], dstv)

    @pl.loop(0, N, step=16)
    def _zero(i):
        counts[pl.ds(i, 16)] = jnp.zeros((16,), jnp.float32)

    ones = jnp.ones((16,), jnp.float32)

    @pl.loop(0, EPT, step=16)
    def _hist(i):
        idx = dstv[pl.ds(i, 16)]
        plsc.addupdate_scatter(counts, [idx], ones)

    pltpu.sync_copy(counts, out_hbm.at[wid])


@functools.lru_cache(maxsize=None)
def _deg_kernel_build():
    return pl.kernel(
        _deg_kernel_body,
        out_type=jax.ShapeDtypeStruct((NW, N), jnp.float32),
        mesh=_mesh(),
        scratch_types=[
            pltpu.VMEM((EPT,), jnp.int32),
            pltpu.VMEM((N,), jnp.float32),
        ],
        compiler_params=_sc_compiler_params(),
    )


def _deg_kernel(dst):
    return _deg_kernel_build()(dst)


def _part_kernel_body(src_hbm, dst_hbm, edges_hbm, counts_hbm,
                      srcv, dstv, bk0, bk1, bk2, bk3, cvv):
    buckets = (bk0, bk1, bk2, bk3)
    c = lax.axis_index("c")
    s = lax.axis_index("s")
    wid = s * NC + c
    pltpu.sync_copy(src_hbm.at[pl.ds(wid * EPT, EPT)], srcv)
    pltpu.sync_copy(dst_hbm.at[pl.ds(wid * EPT, EPT)], dstv)

    zero = jnp.int32(0)

    @pl.loop(0, EPT, step=16, init_carry=(zero, zero, zero, zero))
    def fills(i, carry):
        sv = srcv[pl.ds(i, 16)]
        dv = dstv[pl.ds(i, 16)]
        sh = jnp.where(sv >= NH, 1, 0).astype(jnp.int32)
        dh = jnp.where(dv >= NH, 1, 0).astype(jnp.int32)
        packed = ((sv - sh * NH) << 16) | (dv - dh * NH)
        bkt = dh * 2 + sh
        out = []
        for b in range(NBK):
            m = bkt == b
            plsc.store_compressed(buckets[b].at[pl.ds(carry[b], 16)],
                                  packed, mask=m)
            cnt = jnp.max(plsc.all_reduce_population_count(m))
            out.append(carry[b] + cnt)
        return tuple(out)

    pad = jnp.full((16,), (NH << 16) | NH, jnp.int32)
    lanes = lax.iota(jnp.int32, 16)
    cv = jnp.zeros((16,), jnp.int32)
    for b in range(NBK):
        f = fills[b]
        for q in range(2 * SCH // 16):
            buckets[b][pl.ds(f + q * 16, 16)] = pad
        fpad = ((f + 2 * SCH - 1) // (2 * SCH)) * (2 * SCH)
        cv = jnp.where(lanes == b, fpad, cv)
    cvv[...] = cv
    pltpu.sync_copy(cvv, counts_hbm.at[pl.ds(wid * 16, 16)])
    for b in range(NBK):
        pltpu.sync_copy(buckets[b],
                        edges_hbm.at[pl.ds((wid * NBK + b) * CAP, CAP)])


@functools.lru_cache(maxsize=None)
def _part_kernel_build():
    return pl.kernel(
        _part_kernel_body,
        out_type=(jax.ShapeDtypeStruct((NW * NBK * CAP,), jnp.int32),
                  jax.ShapeDtypeStruct((NW * 16,), jnp.int32)),
        mesh=_mesh(),
        scratch_types=[
            pltpu.VMEM((EPT,), jnp.int32),
            pltpu.VMEM((EPT,), jnp.int32),
            pltpu.VMEM((CAP,), jnp.int32),
            pltpu.VMEM((CAP,), jnp.int32),
            pltpu.VMEM((CAP,), jnp.int32),
            pltpu.VMEM((CAP,), jnp.int32),
            pltpu.VMEM((16,), jnp.int32),
        ],
        compiler_params=_sc_compiler_params(),
    )


def _part_kernel(src, dst):
    return _part_kernel_build()(src, dst)


def _acc_kernel_body(hp_hbm, edges_hbm, counts_hbm, out_hbm,
                     pkvA, pkvB, sidxA, sidxB, didxA, didxB, rows, cvm,
                     hpl, acc, sem, sem_i, sem_s):
    c = lax.axis_index("c")
    s = lax.axis_index("s")
    lanes = lax.iota(jnp.int32, 16)
    ob = s * SUBROWS

    pltpu.sync_copy(counts_hbm, cvm)

    pltpu.sync_copy(hp_hbm.at[pl.ds(c * NH + ob, SUBROWS)],
                    acc.at[pl.ds(ob, SUBROWS)])

    @pl.when(s == 0)
    def _seed_tail():
        pltpu.sync_copy(hp_hbm.at[pl.ds(c * NH + NS * SUBROWS, NLOC - NS * SUBROWS)],
                        acc.at[pl.ds(NS * SUBROWS, NLOC - NS * SUBROWS)])

    for p in range(2):
        pltpu.sync_copy(hp_hbm.at[pl.ds(p * NH + ob, SUBROWS)],
                        hpl.at[pl.ds(ob, SUBROWS)])

        @pl.when(s == 0)
        def _stage_tail():
            pltpu.sync_copy(
                hp_hbm.at[pl.ds(p * NH + NS * SUBROWS, NLOC - NS * SUBROWS)],
                hpl.at[pl.ds(NS * SUBROWS, NLOC - NS * SUBROWS)])

        plsc.subcore_barrier()
        b = c * 2 + p
        for t_off in range(2):
            t = 2 * s + t_off
            crow = cvm[pl.ds(t * 16, 16)]
            cnt = jnp.max(jnp.where(lanes == b, crow, 0))
            regbase = (t * NBK + b) * CAP

            def idx_load(off, buf):
                return pltpu.async_copy(
                    edges_hbm.at[pl.ds(regbase + off, SCH)], buf, sem_i)

            def unpack(pkv, sidx, didx):
                for q in range(SCH // 16):
                    pk = pkv[pl.ds(q * 16, 16)]
                    sidx[pl.ds(q * 16, 16)] = pk >> 16
                    didx[pl.ds(q * 16, 16)] = pk & 0xFFFF

            idx_load(0, pkvA)
            idx_load(SCH, pkvB)

            @pl.loop(0, cnt, step=2 * SCH)
            def _pair(k):
                pltpu.make_async_copy(edges_hbm.at[pl.ds(regbase + k, SCH)],
                                      pkvA, sem_i).wait()
                unpack(pkvA, sidxA, didxA)
                pltpu.async_copy(hpl.at[sidxA], rows.at[0], sem)
                idx_load(k + 2 * SCH, pkvA)
                pltpu.make_async_copy(
                    edges_hbm.at[pl.ds(regbase + k + SCH, SCH)],
                    pkvB, sem_i).wait()
                unpack(pkvB, sidxB, didxB)
                pltpu.make_async_copy(hpl.at[sidxA], rows.at[0], sem).wait()
                pltpu.async_copy(hpl.at[sidxB], rows.at[1], sem)
                pltpu.async_copy(rows.at[0], acc.at[didxA], sem_s, add=True)
                idx_load(k + 3 * SCH, pkvB)
                pltpu.make_async_copy(hpl.at[sidxB], rows.at[1], sem).wait()
                pltpu.sync_copy(rows.at[1], acc.at[didxB], add=True)
                pltpu.make_async_copy(rows.at[0], acc.at[didxA],
                                      sem_s).wait()

            pltpu.make_async_copy(edges_hbm.at[pl.ds(regbase, SCH)],
                                  pkvA, sem_i).wait()
            pltpu.make_async_copy(edges_hbm.at[pl.ds(regbase, SCH)],
                                  pkvB, sem_i).wait()

        plsc.subcore_barrier()

    pltpu.sync_copy(acc.at[pl.ds(ob, SUBROWS)],
                    out_hbm.at[c, pl.ds(ob, SUBROWS)])

    @pl.when(s == 0)
    def _out_tail():
        pltpu.sync_copy(acc.at[pl.ds(NS * SUBROWS, NLOC - NS * SUBROWS)],
                        out_hbm.at[c, pl.ds(NS * SUBROWS, NLOC - NS * SUBROWS)])


@functools.lru_cache(maxsize=None)
def _acc_kernel_build():
    return pl.kernel(
        _acc_kernel_body,
        out_type=jax.ShapeDtypeStruct((NC, NLOC, D), jnp.float32),
        mesh=_mesh(),
        scratch_types=[
            pltpu.VMEM((SCH,), jnp.int32),
            pltpu.VMEM((SCH,), jnp.int32),
            pltpu.VMEM((SCH,), jnp.int32),
            pltpu.VMEM((SCH,), jnp.int32),
            pltpu.VMEM((SCH,), jnp.int32),
            pltpu.VMEM((SCH,), jnp.int32),
            pltpu.VMEM((2, SCH, D), jnp.float32),
            pltpu.VMEM((NW * 16,), jnp.int32),
            pltpu.VMEM_SHARED((NLOC, D), jnp.float32),
            pltpu.VMEM_SHARED((NLOC, D), jnp.float32),
            pltpu.SemaphoreType.DMA,
            pltpu.SemaphoreType.DMA,
            pltpu.SemaphoreType.DMA,
        ],
        compiler_params=_sc_compiler_params(),
    )


def _acc_kernel(hp, edges, counts):
    return _acc_kernel_build()(hp, edges, counts)


_RB = 1000
_RBM = 2000


def _dinv_body(c_ref, o_ref):
    deg = jnp.sum(c_ref[...], axis=0, keepdims=True) + 1.0
    o_ref[...] = lax.rsqrt(deg)


def _mm_scale_body(x_ref, w_ref, dv_ref, o_ref):
    h = jnp.dot(x_ref[...], w_ref[...], preferred_element_type=jnp.float32)
    o_ref[...] = h * dv_ref[...]


def _ln_gelu(a, dv, b, g, be):
    y = a * dv + b
    mu = jnp.mean(y, axis=1, keepdims=True)
    d = y - mu
    var = jnp.mean(d * d, axis=1, keepdims=True)
    t = d * lax.rsqrt(var + 1e-5) * g + be
    return t * 0.5 * (1.0 + lax.erf(t * np.float32(1.0 / np.sqrt(2.0))))


def _comb_mm_body(acc_ref, dv_ref, b_ref, g_ref, be_ref, w_ref, o_ref):
    a = acc_ref[0]
    t = _ln_gelu(a, dv_ref[...], b_ref[...], g_ref[...], be_ref[...])
    h = jnp.dot(t, w_ref[...], preferred_element_type=jnp.float32)
    o_ref[...] = h * dv_ref[...]


def _comb_final_body(acc_ref, dv_ref, b_ref, g_ref, be_ref, o_ref):
    a = acc_ref[0]
    o_ref[...] = _ln_gelu(a, dv_ref[...], b_ref[...], g_ref[...], be_ref[...])


def _acc_spec():
    return pl.BlockSpec((1, _RB, D), lambda i: (i // 5, i % 5, 0))


def _dinv_call(counts):
    return pl.pallas_call(
        _dinv_body,
        out_shape=jax.ShapeDtypeStruct((1, N), jnp.float32),
    )(counts)


def _mm_scale_call(x, w, dv):
    return pl.pallas_call(
        _mm_scale_body,
        grid=(N // _RBM,),
        in_specs=[
            pl.BlockSpec((_RBM, D), lambda i: (i, 0)),
            pl.BlockSpec((D, D), lambda i: (0, 0)),
            pl.BlockSpec((_RBM, 1), lambda i: (i, 0)),
        ],
        out_specs=pl.BlockSpec((_RBM, D), lambda i: (i, 0)),
        out_shape=jax.ShapeDtypeStruct((N_PAD, D), jnp.float32),
    )(x, w, dv)


def _comb_mm_call(accs, dv, b, g, be, w):
    return pl.pallas_call(
        _comb_mm_body,
        grid=(N // _RB,),
        in_specs=[
            _acc_spec(),
            pl.BlockSpec((_RB, 1), lambda i: (i, 0)),
            pl.BlockSpec((1, D), lambda i: (0, 0)),
            pl.BlockSpec((1, D), lambda i: (0, 0)),
            pl.BlockSpec((1, D), lambda i: (0, 0)),
            pl.BlockSpec((D, D), lambda i: (0, 0)),
        ],
        out_specs=pl.BlockSpec((_RB, D), lambda i: (i, 0)),
        out_shape=jax.ShapeDtypeStruct((N_PAD, D), jnp.float32),
    )(accs, dv, b, g, be, w)


def _comb_final_call(accs, dv, b, g, be):
    return pl.pallas_call(
        _comb_final_body,
        grid=(N // _RB,),
        in_specs=[
            _acc_spec(),
            pl.BlockSpec((_RB, 1), lambda i: (i, 0)),
            pl.BlockSpec((1, D), lambda i: (0, 0)),
            pl.BlockSpec((1, D), lambda i: (0, 0)),
            pl.BlockSpec((1, D), lambda i: (0, 0)),
        ],
        out_specs=pl.BlockSpec((_RB, D), lambda i: (i, 0)),
        out_shape=jax.ShapeDtypeStruct((N, D), jnp.float32),
    )(accs, dv, b, g, be)


def kernel(z, edge_index, W0, b0, W1, b1, W2, b2, g0, be0, g1, be1, g2, be2):
    src = edge_index[0]
    dst = edge_index[1]

    counts32 = _deg_kernel(dst)
    edges, counts = _part_kernel(src, dst)
    dv = _dinv_call(counts32).reshape(N, 1)

    b = [b0.reshape(1, D), b1.reshape(1, D), b2.reshape(1, D)]
    g = [g0.reshape(1, D), g1.reshape(1, D), g2.reshape(1, D)]
    be = [be0.reshape(1, D), be1.reshape(1, D), be2.reshape(1, D)]
    Wn = [W1, W2]

    hp = _mm_scale_call(z, W0, dv)
    for i in range(3):
        accs = _acc_kernel(hp, edges, counts)
        if i < 2:
            hp = _comb_mm_call(accs, dv, b[i], g[i], be[i], Wn[i])
        else:
            out = _comb_final_call(accs, dv, b[i], g[i], be[i])
    return out
